# baseline (speedup 1.0000x reference)
import functools

import jax
import jax.numpy as jnp
from jax import lax
from jax.experimental import pallas as pl
from jax.experimental.pallas import tpu as pltpu
from jax.experimental.pallas import tpu_sc as plsc

_B, _L, _D = 4096, 50, 64
_DP = 128
_V = 1000000
_N = _B * _L
_NW = 32
_BPW = _N // _NW
_GR = 320
_G = _BPW // _GR

_CB = 384
_NCH = 999936 // _CB
_XIT = -(-_NCH // _NW)
_TAIL = 999936
_VX = _V + 64

_mesh = plsc.VectorSubcoreMesh(core_axis_name="c", subcore_axis_name="s")
_params = pltpu.CompilerParams(use_tc_tiling_on_sc=True, needs_layout_passes=False)


@functools.partial(
    pl.kernel,
    mesh=_mesh,
    out_type=jax.ShapeDtypeStruct((_VX, _DP), jnp.float32),
    scratch_types=[
        pltpu.VMEM((_D, _CB), jnp.float32),
        pltpu.VMEM((_CB, _DP), jnp.float32),
        pltpu.VMEM((_D, _D), jnp.float32),
    ],
    compiler_params=_params,
)
def _xpose(tt_hbm, tail_hbm, xp_hbm, in_v, out_v, tail_v):
    wid = lax.axis_index("s") * 2 + lax.axis_index("c")
    didx = [lax.iota(jnp.int32, 16) + 16 * cb for cb in range(4)]

    def do_rows(nrows, src_v):
        def row_body(rr, carry):
            rrv = jnp.full((16,), rr, jnp.int32)
            for cb in range(4):
                out_v[rr, pl.ds(cb * 16, 16)] = plsc.load_gather(
                    src_v, [didx[cb], rrv])
            return carry
        lax.fori_loop(0, nrows, row_body, 0)

    def chunk_body(i, carry):
        cid = i * _NW + wid
        @pl.when(cid < _NCH)
        def _():
            c0 = cid * _CB
            pltpu.sync_copy(tt_hbm.at[:, pl.ds(c0, _CB)], in_v)
            do_rows(_CB, in_v)
            pltpu.sync_copy(out_v, xp_hbm.at[pl.ds(c0, _CB)])
        return carry

    lax.fori_loop(0, _XIT, chunk_body, 0)

    @pl.when(wid == 0)
    def _():
        pltpu.sync_copy(tail_hbm, tail_v)
        do_rows(64, tail_v)
        pltpu.sync_copy(out_v.at[pl.ds(0, 64)], xp_hbm.at[pl.ds(_V, 64)])


@functools.partial(
    pl.kernel,
    mesh=_mesh,
    out_type=jax.ShapeDtypeStruct((_N, _DP), jnp.float32),
    scratch_types=[
        pltpu.VMEM((_BPW,), jnp.int32),
        pltpu.VMEM((_GR, _DP), jnp.float32),
        pltpu.VMEM((_GR, _DP), jnp.float32),
        pltpu.SemaphoreType.DMA,
        pltpu.SemaphoreType.DMA,
    ],
    compiler_params=_params,
)
def _gather(table_hbm, idx_hbm, out_hbm, idx_v, rows0, rows1, sem0, sem1):
    wid = lax.axis_index("s") * 2 + lax.axis_index("c")
    base = wid * _BPW
    pltpu.sync_copy(idx_hbm.at[wid], idx_v)

    bufs = (rows0, rows1)
    sems = (sem0, sem1)
    copies = [None, None]
    for g in range(_G):
        b = g % 2
        copies[b] = pltpu.async_copy(
            table_hbm.at[idx_v.at[pl.ds(g * _GR, _GR)]], bufs[b], sems[b])
        if g >= 1:
            pb = (g - 1) % 2
            copies[pb].wait()
            pltpu.sync_copy(bufs[pb], out_hbm.at[pl.ds(base + (g - 1) * _GR, _GR)])
    lb = (_G - 1) % 2
    copies[lb].wait()
    pltpu.sync_copy(bufs[lb], out_hbm.at[pl.ds(base + (_G - 1) * _GR, _GR)])


def kernel(source, table):
    tt = table.T
    xp = _xpose(tt, lax.slice(tt, (0, _TAIL), (_D, _V)))
    idx = source.reshape(_NW, _BPW)
    idx = jnp.where(idx >= _TAIL, idx + 64, idx)
    out = _gather(xp, idx)
    return lax.slice(out, (0, 0), (_N, _D)).reshape(_B, _L, _D)


# speedup vs baseline: 2.1096x; 2.1096x over previous
import functools

import jax
import jax.numpy as jnp
from jax import lax
from jax.experimental import pallas as pl
from jax.experimental.pallas import tpu as pltpu
from jax.experimental.pallas import tpu_sc as plsc

_B, _L, _D = 4096, 50, 64
_DP = 128
_V = 1000000
_N = _B * _L
_NW = 32
_BPW = _N // _NW
_GR = 320
_G = _BPW // _GR

_CB = 256
_NCH = 999936 // _CB
_FULL = _NCH // _NW
_REM = _NCH - _FULL * _NW
_PAIRS = _FULL // 2
_TAIL = 999936
_VX = _V + 64

_mesh = plsc.VectorSubcoreMesh(core_axis_name="c", subcore_axis_name="s")
_params = pltpu.CompilerParams(use_tc_tiling_on_sc=True, needs_layout_passes=False)


@functools.partial(
    pl.kernel,
    mesh=_mesh,
    out_type=jax.ShapeDtypeStruct((_VX, _DP), jnp.float32),
    scratch_types=[
        pltpu.VMEM((_D, _CB), jnp.float32),
        pltpu.VMEM((_D, _CB), jnp.float32),
        pltpu.VMEM((_CB, _DP), jnp.float32),
        pltpu.VMEM((_CB, _DP), jnp.float32),
        pltpu.VMEM((_D, _D), jnp.float32),
        pltpu.SemaphoreType.DMA,
        pltpu.SemaphoreType.DMA,
        pltpu.SemaphoreType.DMA,
        pltpu.SemaphoreType.DMA,
    ],
    compiler_params=_params,
)
def _xpose(tt_hbm, tail_hbm, xp_hbm, in0, in1, ob0, ob1, tail_v,
           si0, si1, so0, so1):
    wid = lax.axis_index("s") * 2 + lax.axis_index("c")
    iot = lax.iota(jnp.int32, 16)
    perms = [(iot + s) & 15 for s in range(16)]
    base = _FULL * wid + jnp.minimum(wid, _REM)

    ins = (in0, in1)
    obs = (ob0, ob1)
    sis = (si0, si1)
    sos = (so0, so1)

    def start_in(k, b):
        pltpu.async_copy(tt_hbm.at[:, pl.ds((base + k) * _CB, _CB)],
                         ins[b], sis[b])

    def wait_in(b):
        pltpu.make_async_copy(tt_hbm.at[:, pl.ds(0, _CB)], ins[b], sis[b]).wait()

    def start_out(k, b):
        pltpu.async_copy(obs[b], xp_hbm.at[pl.ds((base + k) * _CB, _CB)], sos[b])

    def wait_out(b):
        pltpu.make_async_copy(obs[b], xp_hbm.at[pl.ds(0, _CB)], sos[b]).wait()

    def compute(in_b, ob_b):
        def blk(rr0, carry):
            rrv = iot + rr0 * 16
            for cb in range(4):
                for s in range(16):
                    d_idx = perms[s] + (16 * cb)
                    v = plsc.load_gather(in_b, [d_idx, rrv])
                    plsc.store_scatter(ob_b, [rrv, d_idx], v)
            return carry
        lax.fori_loop(0, _CB // 16, blk, 0)

    start_in(0, 0)

    def pair_body(p, carry):
        for b in range(2):
            k = p * 2 + b
            wait_in(b)
            if b == 0:
                start_in(k + 1, 1)
            else:
                @pl.when(p < _PAIRS - 1)
                def _():
                    start_in(k + 1, 0)
            @pl.when(p >= 1)
            def _():
                wait_out(b)
            compute(ins[b], obs[b])
            start_out(k, b)
        return carry

    lax.fori_loop(0, _PAIRS, pair_body, 0)
    wait_out(0)
    wait_out(1)

    @pl.when(wid < _REM)
    def _():
        k = _FULL
        pltpu.sync_copy(tt_hbm.at[:, pl.ds((base + k) * _CB, _CB)], in0)
        compute(in0, ob0)
        pltpu.sync_copy(ob0, xp_hbm.at[pl.ds((base + k) * _CB, _CB)])

    @pl.when(wid == _NW - 1)
    def _():
        pltpu.sync_copy(tail_hbm, tail_v)
        def row_body(rr, carry):
            rrv = jnp.full((16,), rr, jnp.int32)
            for cb in range(4):
                ob1[rr, pl.ds(cb * 16, 16)] = plsc.load_gather(
                    tail_v, [iot + 16 * cb, rrv])
            return carry
        lax.fori_loop(0, 64, row_body, 0)
        pltpu.sync_copy(ob1.at[pl.ds(0, 64)], xp_hbm.at[pl.ds(_V, 64)])


@functools.partial(
    pl.kernel,
    mesh=_mesh,
    out_type=jax.ShapeDtypeStruct((_N, _DP), jnp.float32),
    scratch_types=[
        pltpu.VMEM((_BPW,), jnp.int32),
        pltpu.VMEM((_GR, _DP), jnp.float32),
        pltpu.VMEM((_GR, _DP), jnp.float32),
        pltpu.SemaphoreType.DMA,
        pltpu.SemaphoreType.DMA,
    ],
    compiler_params=_params,
)
def _gather(table_hbm, idx_hbm, out_hbm, idx_v, rows0, rows1, sem0, sem1):
    wid = lax.axis_index("s") * 2 + lax.axis_index("c")
    base = wid * _BPW
    pltpu.sync_copy(idx_hbm.at[wid], idx_v)

    bufs = (rows0, rows1)
    sems = (sem0, sem1)
    copies = [None, None]
    for g in range(_G):
        b = g % 2
        copies[b] = pltpu.async_copy(
            table_hbm.at[idx_v.at[pl.ds(g * _GR, _GR)]], bufs[b], sems[b])
        if g >= 1:
            pb = (g - 1) % 2
            copies[pb].wait()
            pltpu.sync_copy(bufs[pb], out_hbm.at[pl.ds(base + (g - 1) * _GR, _GR)])
    lb = (_G - 1) % 2
    copies[lb].wait()
    pltpu.sync_copy(bufs[lb], out_hbm.at[pl.ds(base + (_G - 1) * _GR, _GR)])


def kernel(source, table):
    tt = table.T
    xp = _xpose(tt, lax.slice(tt, (0, _TAIL), (_D, _V)))
    idx = source.reshape(_NW, _BPW)
    idx = jnp.where(idx >= _TAIL, idx + 64, idx)
    out = _gather(xp, idx)
    return lax.slice(out, (0, 0), (_N, _D)).reshape(_B, _L, _D)


# speedup vs baseline: 3.8375x; 1.8191x over previous
import functools

import jax
import jax.numpy as jnp
from jax import lax
from jax.experimental import pallas as pl
from jax.experimental.pallas import tpu as pltpu
from jax.experimental.pallas import tpu_sc as plsc

_B, _L, _D = 4096, 50, 64
_DP = 128
_V = 1000000
_N = _B * _L
_NW = 32
_BPW = _N // _NW
_GR = 320
_G = _BPW // _GR

_CB = 256
_NCH = 999936 // _CB
_FULL = _NCH // _NW
_REM = _NCH - _FULL * _NW
_PAIRS = _FULL // 2
_TAIL = 999936
_VX = _V + 64

_mesh = plsc.VectorSubcoreMesh(core_axis_name="c", subcore_axis_name="s")
_params = pltpu.CompilerParams(use_tc_tiling_on_sc=True, needs_layout_passes=False)


@functools.partial(
    pl.kernel,
    mesh=_mesh,
    out_type=jax.ShapeDtypeStruct((_VX, _DP), jnp.float32),
    scratch_types=[
        pltpu.VMEM((_D, _CB), jnp.float32),
        pltpu.VMEM((_D, _CB), jnp.float32),
        pltpu.VMEM((_CB, _DP), jnp.float32),
        pltpu.VMEM((_CB, _DP), jnp.float32),
        pltpu.VMEM((_D, _D), jnp.float32),
        pltpu.SemaphoreType.DMA,
        pltpu.SemaphoreType.DMA,
        pltpu.SemaphoreType.DMA,
        pltpu.SemaphoreType.DMA,
    ],
    compiler_params=_params,
)
def _xpose(tt_hbm, tail_hbm, xp_hbm, in0, in1, ob0, ob1, tail_v,
           si0, si1, so0, so1):
    wid = lax.axis_index("s") * 2 + lax.axis_index("c")
    iot = lax.iota(jnp.int32, 16)
    perms = [(iot + s) & 15 for s in range(16)]
    base = _FULL * wid + jnp.minimum(wid, _REM)

    ins = (in0, in1)
    obs = (ob0, ob1)
    sis = (si0, si1)
    sos = (so0, so1)

    def start_in(k, b):
        pltpu.async_copy(tt_hbm.at[:, pl.ds((base + k) * _CB, _CB)],
                         ins[b], sis[b])

    def wait_in(b):
        pltpu.make_async_copy(tt_hbm.at[:, pl.ds(0, _CB)], ins[b], sis[b]).wait()

    def start_out(k, b):
        pltpu.async_copy(obs[b], xp_hbm.at[pl.ds((base + k) * _CB, _CB)], sos[b])

    def wait_out(b):
        pltpu.make_async_copy(obs[b], xp_hbm.at[pl.ds(0, _CB)], sos[b]).wait()

    def compute(in_b, ob_b):
        def blk(rr0, carry):
            rrv = iot + rr0 * 16
            for cb in range(4):
                dss = [perms[s] + (16 * cb) for s in range(16)]
                vals = [plsc.load_gather(in_b, [dss[s], rrv])
                        for s in range(16)]
                for s in range(16):
                    plsc.store_scatter(ob_b, [rrv, dss[s]], vals[s])
            return carry
        lax.fori_loop(0, _CB // 16, blk, 0)

    start_in(0, 0)

    def pair_body(p, carry):
        for b in range(2):
            k = p * 2 + b
            wait_in(b)
            if b == 0:
                start_in(k + 1, 1)
            else:
                @pl.when(p < _PAIRS - 1)
                def _():
                    start_in(k + 1, 0)
            @pl.when(p >= 1)
            def _():
                wait_out(b)
            compute(ins[b], obs[b])
            start_out(k, b)
        return carry

    lax.fori_loop(0, _PAIRS, pair_body, 0)
    wait_out(0)
    wait_out(1)

    @pl.when(wid < _REM)
    def _():
        k = _FULL
        pltpu.sync_copy(tt_hbm.at[:, pl.ds((base + k) * _CB, _CB)], in0)
        compute(in0, ob0)
        pltpu.sync_copy(ob0, xp_hbm.at[pl.ds((base + k) * _CB, _CB)])

    @pl.when(wid == _NW - 1)
    def _():
        pltpu.sync_copy(tail_hbm, tail_v)
        def row_body(rr, carry):
            rrv = jnp.full((16,), rr, jnp.int32)
            for cb in range(4):
                ob1[rr, pl.ds(cb * 16, 16)] = plsc.load_gather(
                    tail_v, [iot + 16 * cb, rrv])
            return carry
        lax.fori_loop(0, 64, row_body, 0)
        pltpu.sync_copy(ob1.at[pl.ds(0, 64)], xp_hbm.at[pl.ds(_V, 64)])


@functools.partial(
    pl.kernel,
    mesh=_mesh,
    out_type=jax.ShapeDtypeStruct((_N, _DP), jnp.float32),
    scratch_types=[
        pltpu.VMEM((_BPW,), jnp.int32),
        pltpu.VMEM((_GR, _DP), jnp.float32),
        pltpu.VMEM((_GR, _DP), jnp.float32),
        pltpu.SemaphoreType.DMA,
        pltpu.SemaphoreType.DMA,
    ],
    compiler_params=_params,
)
def _gather(table_hbm, idx_hbm, out_hbm, idx_v, rows0, rows1, sem0, sem1):
    wid = lax.axis_index("s") * 2 + lax.axis_index("c")
    base = wid * _BPW
    pltpu.sync_copy(idx_hbm.at[wid], idx_v)

    bufs = (rows0, rows1)
    sems = (sem0, sem1)
    copies = [None, None]
    for g in range(_G):
        b = g % 2
        copies[b] = pltpu.async_copy(
            table_hbm.at[idx_v.at[pl.ds(g * _GR, _GR)]], bufs[b], sems[b])
        if g >= 1:
            pb = (g - 1) % 2
            copies[pb].wait()
            pltpu.sync_copy(bufs[pb], out_hbm.at[pl.ds(base + (g - 1) * _GR, _GR)])
    lb = (_G - 1) % 2
    copies[lb].wait()
    pltpu.sync_copy(bufs[lb], out_hbm.at[pl.ds(base + (_G - 1) * _GR, _GR)])


def kernel(source, table):
    tt = table.T
    xp = _xpose(tt, lax.slice(tt, (0, _TAIL), (_D, _V)))
    idx = source.reshape(_NW, _BPW)
    idx = jnp.where(idx >= _TAIL, idx + 64, idx)
    out = _gather(xp, idx)
    return lax.slice(out, (0, 0), (_N, _D)).reshape(_B, _L, _D)
